# baseline (speedup 1.0000x reference)
import jax
import jax.numpy as jnp
from jax import lax
from jax.experimental import pallas as pl
from jax.experimental.pallas import tpu as pltpu
from jax.experimental.pallas import tpu_sc as plsc

E = 8
K = 2
TOKENS = 4096
D = 2048
F = 2048
TM = 256
P = TOKENS * K
PAD_ROWS = P + E * TM
NUM_TILES = PAD_ROWS // TM

NW = 32
TOK_W = TOKENS // NW
CTD = 16
GD = TOK_W // CTD // 2
CR = 16
NCHZ = 2 * TOK_W // CR
TB = 512

_SC_MESH = plsc.VectorSubcoreMesh(core_axis_name="c", subcore_axis_name="s")


def _worker_id():
    return lax.axis_index("s") * 2 + lax.axis_index("c")


def _dispatch_body(x_hbm, de_hbm, do_hbm, xs_hbm, xbuf, ie, io, lsem0, lsem1, ssem):
    wid = _worker_id()
    base = wid * TOK_W
    crow = wid * (2 * GD)
    lsems = (lsem0, lsem1)

    pltpu.sync_copy(de_hbm.at[pl.ds(crow, 2 * GD)], ie)
    pltpu.sync_copy(do_hbm.at[pl.ds(crow, 2 * GD)], io)

    def start_load(i, slot):
        off = base + jnp.minimum(i, 2 * GD - 1) * CTD
        pltpu.async_copy(x_hbm.at[pl.ds(off, CTD)], xbuf.at[slot], lsems[slot])

    def wait_load(slot):
        pltpu.make_async_copy(x_hbm.at[pl.ds(0, CTD)], xbuf.at[slot],
                              lsems[slot]).wait()

    def scatter(i, slot):
        h0 = pltpu.async_copy(xbuf.at[slot], xs_hbm.at[ie.at[i]], ssem)
        h1 = pltpu.async_copy(xbuf.at[slot], xs_hbm.at[io.at[i]], ssem)
        h0.wait()
        h1.wait()

    start_load(0, 0)

    def round_(g, carry):
        start_load(2 * g + 1, 1)
        wait_load(0)
        scatter(2 * g, 0)
        start_load(2 * g + 2, 0)
        wait_load(1)
        scatter(2 * g + 1, 1)
        return carry

    lax.fori_loop(0, GD, round_, 0)
    wait_load(0)


def _zgather_body(ys_hbm, dest_hbm, z_hbm, pbuf, ip,
                  gsem0, gsem1, wsem0, wsem1):
    wid = _worker_id()
    base_row = wid * (2 * TOK_W)
    gsems = (gsem0, gsem1)
    wsems = (wsem0, wsem1)

    pltpu.sync_copy(dest_hbm.at[pl.ds(wid * NCHZ, NCHZ)], ip)

    def start_gather(i, slot):
        ic = jnp.minimum(i, NCHZ - 1)
        pltpu.async_copy(ys_hbm.at[ip.at[ic]], pbuf.at[slot], gsems[slot])

    def wait_gather(slot):
        pltpu.make_async_copy(ys_hbm.at[ip.at[0]], pbuf.at[slot],
                              gsems[slot]).wait()

    def write(i, slot):
        off = base_row + i * CR
        pltpu.async_copy(pbuf.at[slot], z_hbm.at[pl.ds(off, CR)],
                         wsems[slot]).wait()

    start_gather(0, 0)

    def round_(g, carry):
        start_gather(2 * g + 1, 1)
        wait_gather(0)
        write(2 * g, 0)
        start_gather(2 * g + 2, 0)
        wait_gather(1)
        write(2 * g + 1, 1)
        return carry

    lax.fori_loop(0, NCHZ // 2, round_, 0)
    wait_gather(0)


def _gate_add_body(z_ref, g0_ref, g1_ref, out_ref):
    z = z_ref[...]
    out_ref[...] = z[:, :D] * g0_ref[0] + z[:, D:] * g1_ref[0]


def _gate_add(z, g0r, g1r):
    return pl.pallas_call(
        _gate_add_body,
        grid=(TOKENS // TB,),
        in_specs=[
            pl.BlockSpec((TB, 2 * D), lambda i: (i, 0)),
            pl.BlockSpec((1, TB, 1), lambda i: (i, 0, 0)),
            pl.BlockSpec((1, TB, 1), lambda i: (i, 0, 0)),
        ],
        out_specs=pl.BlockSpec((TB, D), lambda i: (i, 0)),
        out_shape=jax.ShapeDtypeStruct((TOKENS, D), jnp.float32),
    )(z, g0r, g1r)


def _fc1_body(te_ref, xs_ref, w1_ref, b1_ref, h_ref):
    a = lax.dot_general(xs_ref[...], w1_ref[0], (((1,), (0,)), ((), ())),
                        preferred_element_type=jnp.float32)
    h_ref[...] = jnp.maximum(a + b1_ref[0], 0.0).astype(jnp.bfloat16)


def _fc2_body(te_ref, h_ref, w2_ref, b2_ref, out_ref):
    o = lax.dot_general(h_ref[...], w2_ref[0], (((1,), (0,)), ((), ())),
                        preferred_element_type=jnp.float32)
    out_ref[...] = o + b2_ref[0]


def _grouped_mlp(tile_expert, xs, W1, b1r, W2, b2r):
    fc1_spec = pltpu.PrefetchScalarGridSpec(
        num_scalar_prefetch=1,
        grid=(NUM_TILES,),
        in_specs=[
            pl.BlockSpec((TM, D), lambda i, te: (i, 0)),
            pl.BlockSpec((1, D, F), lambda i, te: (te[i], 0, 0)),
            pl.BlockSpec((1, 1, F), lambda i, te: (te[i], 0, 0)),
        ],
        out_specs=pl.BlockSpec((TM, F), lambda i, te: (i, 0)),
    )
    h = pl.pallas_call(
        _fc1_body,
        grid_spec=fc1_spec,
        out_shape=jax.ShapeDtypeStruct((PAD_ROWS, F), jnp.bfloat16),
        compiler_params=pltpu.CompilerParams(
            dimension_semantics=("arbitrary",)),
    )(tile_expert, xs, W1, b1r)
    fc2_spec = pltpu.PrefetchScalarGridSpec(
        num_scalar_prefetch=1,
        grid=(NUM_TILES,),
        in_specs=[
            pl.BlockSpec((TM, F), lambda i, te: (i, 0)),
            pl.BlockSpec((1, F, D), lambda i, te: (te[i], 0, 0)),
            pl.BlockSpec((1, 1, D), lambda i, te: (te[i], 0, 0)),
        ],
        out_specs=pl.BlockSpec((TM, D), lambda i, te: (i, 0)),
    )
    return pl.pallas_call(
        _fc2_body,
        grid_spec=fc2_spec,
        out_shape=jax.ShapeDtypeStruct((PAD_ROWS, D), jnp.float32),
        compiler_params=pltpu.CompilerParams(
            dimension_semantics=("arbitrary",)),
    )(tile_expert, h, W2, b2r)


def kernel(x, topK_indices, topK_scores, W1, b1, W2, b2):
    flat_e = topK_indices.reshape(-1).astype(jnp.int32)
    flat_g = topK_scores.reshape(-1).astype(jnp.float32)

    oh = (flat_e[:, None] == jnp.arange(E, dtype=jnp.int32)[None, :]).astype(jnp.int32)
    cum = jnp.cumsum(oh, axis=0)
    sizes = cum[-1]
    rank = jnp.sum(oh * cum, axis=1) - 1
    tiles_per = (sizes + TM - 1) // TM
    pstarts = (jnp.cumsum(tiles_per) - tiles_per) * TM
    dest = (jnp.sum(oh * pstarts[None, :], axis=1) + rank).astype(jnp.int32)

    tile_expert = jnp.repeat(jnp.arange(E, dtype=jnp.int32), tiles_per,
                             total_repeat_length=NUM_TILES)
    d_even = dest[0::2]
    d_odd = dest[1::2]

    xs = pl.kernel(
        _dispatch_body,
        mesh=_SC_MESH,
        out_type=jax.ShapeDtypeStruct((PAD_ROWS, D), jnp.float32),
        scratch_types=[
            pltpu.VMEM((2, CTD, D), jnp.float32),
            pltpu.VMEM((2 * GD, CTD), jnp.int32),
            pltpu.VMEM((2 * GD, CTD), jnp.int32),
            pltpu.SemaphoreType.DMA,
            pltpu.SemaphoreType.DMA,
            pltpu.SemaphoreType.DMA,
        ],
    )(x, d_even.reshape(TOKENS // CTD, CTD), d_odd.reshape(TOKENS // CTD, CTD))

    ys = _grouped_mlp(
        tile_expert,
        xs,
        W1,
        b1.reshape(E, 1, F),
        W2,
        b2.reshape(E, 1, D),
    )

    z = pl.kernel(
        _zgather_body,
        mesh=_SC_MESH,
        out_type=jax.ShapeDtypeStruct((P, D), jnp.float32),
        scratch_types=[
            pltpu.VMEM((2, CR, D), jnp.float32),
            pltpu.VMEM((NCHZ, CR), jnp.int32),
            pltpu.SemaphoreType.DMA,
            pltpu.SemaphoreType.DMA,
            pltpu.SemaphoreType.DMA,
            pltpu.SemaphoreType.DMA,
        ],
    )(ys, dest.reshape(P // CR, CR))

    return _gate_add(
        z.reshape(TOKENS, 2 * D),
        topK_scores[:, 0].astype(jnp.float32).reshape(TOKENS // TB, TB, 1),
        topK_scores[:, 1].astype(jnp.float32).reshape(TOKENS // TB, TB, 1),
    )


# speedup vs baseline: 1.0787x; 1.0787x over previous
import jax
import jax.numpy as jnp
from jax import lax
from jax.experimental import pallas as pl
from jax.experimental.pallas import tpu as pltpu
from jax.experimental.pallas import tpu_sc as plsc

E = 8
K = 2
TOKENS = 4096
D = 2048
F = 2048
TM = 256
P = TOKENS * K
PAD_ROWS = P + E * TM
NUM_TILES = PAD_ROWS // TM

NW = 32
TOK_W = TOKENS // NW
CTD = 16
GD = TOK_W // CTD // 2
CTC = 8
GC = TOK_W // CTC // 2
UN = 16

_SC_MESH = plsc.VectorSubcoreMesh(core_axis_name="c", subcore_axis_name="s")


def _worker_id():
    return lax.axis_index("s") * 2 + lax.axis_index("c")


def _dispatch_body(x_hbm, de_hbm, do_hbm, xs_hbm, xbuf, ie, io, lsem0, lsem1, ssem):
    wid = _worker_id()
    base = wid * TOK_W
    crow = wid * (2 * GD)
    lsems = (lsem0, lsem1)

    pltpu.sync_copy(de_hbm.at[pl.ds(crow, 2 * GD)], ie)
    pltpu.sync_copy(do_hbm.at[pl.ds(crow, 2 * GD)], io)

    def start_load(i, slot):
        off = base + jnp.minimum(i, 2 * GD - 1) * CTD
        pltpu.async_copy(x_hbm.at[pl.ds(off, CTD)], xbuf.at[slot], lsems[slot])

    def wait_load(slot):
        pltpu.make_async_copy(x_hbm.at[pl.ds(0, CTD)], xbuf.at[slot],
                              lsems[slot]).wait()

    def scatter(i, slot):
        h0 = pltpu.async_copy(xbuf.at[slot], xs_hbm.at[ie.at[i]], ssem)
        h1 = pltpu.async_copy(xbuf.at[slot], xs_hbm.at[io.at[i]], ssem)
        h0.wait()
        h1.wait()

    start_load(0, 0)

    def round_(g, carry):
        start_load(2 * g + 1, 1)
        wait_load(0)
        scatter(2 * g, 0)
        start_load(2 * g + 2, 0)
        wait_load(1)
        scatter(2 * g + 1, 1)
        return carry

    lax.fori_loop(0, GD, round_, 0)
    wait_load(0)


def _combine_body(ys_hbm, dest_hbm, g_hbm, y_hbm, pbuf, obuf, ip, gbuf,
                  gsem0, gsem1, wsem0, wsem1):
    wid = _worker_id()
    base = wid * TOK_W
    crow = wid * (2 * GC)
    gsems = (gsem0, gsem1)
    wsems = (wsem0, wsem1)

    pltpu.sync_copy(dest_hbm.at[pl.ds(crow, 2 * GC)], ip)
    pltpu.sync_copy(g_hbm.at[pl.ds(crow, 2 * GC)], gbuf)

    def start_gather(i, slot):
        ic = jnp.minimum(i, 2 * GC - 1)
        pltpu.async_copy(ys_hbm.at[ip.at[ic]], pbuf.at[slot], gsems[slot])

    def wait_gather(slot):
        pltpu.make_async_copy(ys_hbm.at[ip.at[0]], pbuf.at[slot],
                              gsems[slot]).wait()

    def wait_write(i, slot):
        pltpu.make_async_copy(obuf.at[slot], y_hbm.at[pl.ds(0, CTC)],
                              wsems[slot]).wait()

    def compute_and_write(i, slot):
        def row(r, c2):
            g0 = gbuf[i, pl.ds(2 * r * 16, 16)]
            g1 = gbuf[i, pl.ds((2 * r + 1) * 16, 16)]

            def inner(c, c3):
                for u in range(UN):
                    sl = pl.ds((c * UN + u) * 16, 16)
                    obuf[slot, r, sl] = (pbuf[slot, 2 * r, sl] * g0
                                         + pbuf[slot, 2 * r + 1, sl] * g1)
                return c3

            return lax.fori_loop(0, (D // 16) // UN, inner, c2)

        lax.fori_loop(0, CTC, row, 0)
        off = base + i * CTC
        pltpu.async_copy(obuf.at[slot], y_hbm.at[pl.ds(off, CTC)], wsems[slot])

    start_gather(0, 0)

    def round_(g, carry):
        start_gather(2 * g + 1, 1)
        wait_gather(0)

        @pl.when(g >= 1)
        def _():
            wait_write(2 * g, 0)

        compute_and_write(2 * g, 0)
        start_gather(2 * g + 2, 0)
        wait_gather(1)

        @pl.when(g >= 1)
        def _():
            wait_write(2 * g + 1, 1)

        compute_and_write(2 * g + 1, 1)
        return carry

    lax.fori_loop(0, GC, round_, 0)
    wait_gather(0)
    wait_write(0, 0)
    wait_write(0, 1)


def _fc1_body(te_ref, xs_ref, w1_ref, b1_ref, h_ref):
    a = lax.dot_general(xs_ref[...], w1_ref[0], (((1,), (0,)), ((), ())),
                        preferred_element_type=jnp.float32)
    h_ref[...] = jnp.maximum(a + b1_ref[0], 0.0).astype(jnp.bfloat16)


def _fc2_body(te_ref, h_ref, w2_ref, b2_ref, out_ref):
    o = lax.dot_general(h_ref[...], w2_ref[0], (((1,), (0,)), ((), ())),
                        preferred_element_type=jnp.float32)
    out_ref[...] = o + b2_ref[0]


def _grouped_mlp(tile_expert, xs, W1, b1r, W2, b2r):
    fc1_spec = pltpu.PrefetchScalarGridSpec(
        num_scalar_prefetch=1,
        grid=(NUM_TILES,),
        in_specs=[
            pl.BlockSpec((TM, D), lambda i, te: (i, 0)),
            pl.BlockSpec((1, D, F), lambda i, te: (te[i], 0, 0)),
            pl.BlockSpec((1, 1, F), lambda i, te: (te[i], 0, 0)),
        ],
        out_specs=pl.BlockSpec((TM, F), lambda i, te: (i, 0)),
    )
    h = pl.pallas_call(
        _fc1_body,
        grid_spec=fc1_spec,
        out_shape=jax.ShapeDtypeStruct((PAD_ROWS, F), jnp.bfloat16),
        compiler_params=pltpu.CompilerParams(
            dimension_semantics=("arbitrary",)),
    )(tile_expert, xs, W1, b1r)
    fc2_spec = pltpu.PrefetchScalarGridSpec(
        num_scalar_prefetch=1,
        grid=(NUM_TILES,),
        in_specs=[
            pl.BlockSpec((TM, F), lambda i, te: (i, 0)),
            pl.BlockSpec((1, F, D), lambda i, te: (te[i], 0, 0)),
            pl.BlockSpec((1, 1, D), lambda i, te: (te[i], 0, 0)),
        ],
        out_specs=pl.BlockSpec((TM, D), lambda i, te: (i, 0)),
    )
    return pl.pallas_call(
        _fc2_body,
        grid_spec=fc2_spec,
        out_shape=jax.ShapeDtypeStruct((PAD_ROWS, D), jnp.float32),
        compiler_params=pltpu.CompilerParams(
            dimension_semantics=("arbitrary",)),
    )(tile_expert, h, W2, b2r)


def kernel(x, topK_indices, topK_scores, W1, b1, W2, b2):
    flat_e = topK_indices.reshape(-1).astype(jnp.int32)
    flat_g = topK_scores.reshape(-1).astype(jnp.float32)

    oh = (flat_e[:, None] == jnp.arange(E, dtype=jnp.int32)[None, :]).astype(jnp.int32)
    cum = jnp.cumsum(oh, axis=0)
    sizes = cum[-1]
    rank = jnp.sum(oh * cum, axis=1) - 1
    tiles_per = (sizes + TM - 1) // TM
    pstarts = (jnp.cumsum(tiles_per) - tiles_per) * TM
    dest = (jnp.sum(oh * pstarts[None, :], axis=1) + rank).astype(jnp.int32)

    tile_expert = jnp.repeat(jnp.arange(E, dtype=jnp.int32), tiles_per,
                             total_repeat_length=NUM_TILES)
    d_even = dest[0::2]
    d_odd = dest[1::2]

    xs = pl.kernel(
        _dispatch_body,
        mesh=_SC_MESH,
        out_type=jax.ShapeDtypeStruct((PAD_ROWS, D), jnp.float32),
        scratch_types=[
            pltpu.VMEM((2, CTD, D), jnp.float32),
            pltpu.VMEM((2 * GD, CTD), jnp.int32),
            pltpu.VMEM((2 * GD, CTD), jnp.int32),
            pltpu.SemaphoreType.DMA,
            pltpu.SemaphoreType.DMA,
            pltpu.SemaphoreType.DMA,
        ],
    )(x, d_even.reshape(TOKENS // CTD, CTD), d_odd.reshape(TOKENS // CTD, CTD))

    ys = _grouped_mlp(
        tile_expert,
        xs,
        W1,
        b1.reshape(E, 1, F),
        W2,
        b2.reshape(E, 1, D),
    )

    y = pl.kernel(
        _combine_body,
        mesh=_SC_MESH,
        out_type=jax.ShapeDtypeStruct((TOKENS, D), jnp.float32),
        scratch_types=[
            pltpu.VMEM((2, 2 * CTC, D), jnp.float32),
            pltpu.VMEM((2, CTC, D), jnp.float32),
            pltpu.VMEM((2 * GC, 2 * CTC), jnp.int32),
            pltpu.VMEM((2 * GC, 2 * CTC * 16), jnp.float32),
            pltpu.SemaphoreType.DMA,
            pltpu.SemaphoreType.DMA,
            pltpu.SemaphoreType.DMA,
            pltpu.SemaphoreType.DMA,
        ],
    )(ys, dest.reshape(TOKENS // CTC, 2 * CTC),
      jnp.broadcast_to(flat_g[:, None], (P, 16)).reshape(TOKENS // CTC, 2 * CTC * 16))
    return y
